# baseline (speedup 1.0000x reference)
import jax
import jax.numpy as jnp
from jax.experimental import pallas as pl
from jax.experimental.pallas import tpu as pltpu

B_, S_, H_, E_, K_ = 4, 8192, 1024, 64, 8
ALPHA_ = 0.1
TOK_BLK = 4096
AUX_SCALE = E_ * ALPHA_ / (B_ * float(S_) * float(S_) * K_)


def _gate_kernel(x_ref, w_ref, out_ref, aux_ref, acc_ref, cnt_ref):
    step = pl.program_id(0)

    @pl.when(step == 0)
    def _init():
        acc_ref[...] = jnp.zeros_like(acc_ref)
        cnt_ref[...] = jnp.zeros_like(cnt_ref)

    x = x_ref[...]
    w = w_ref[...]
    logits = jax.lax.dot_general(
        w, x, (((1,), (1,)), ((), ())), preferred_element_type=jnp.float32
    )
    m = jnp.max(logits, axis=0, keepdims=True)
    ex = jnp.exp(logits - m)
    z = jnp.sum(ex, axis=0, keepdims=True)
    scores = ex / z

    iota = jax.lax.broadcasted_iota(jnp.int32, scores.shape, 0).astype(
        jnp.float32
    )
    s = scores
    idx_rows = []
    val_rows = []
    for k in range(K_):
        if k == 0:
            vmax = 1.0 / z
        else:
            vmax = jnp.max(s, axis=0, keepdims=True)
        cand = jnp.where(s == vmax, iota, 64.0)
        imax = jnp.min(cand, axis=0, keepdims=True)
        onehot = cand == imax
        s = jnp.where(onehot, -1.0, s)
        idx_rows.append(imax)
        val_rows.append(vmax)

    vals = jnp.concatenate(val_rows, axis=0)
    denom = jnp.sum(vals, axis=0, keepdims=True) + 1e-20
    out_ref[...] = jnp.concatenate([vals / denom] + idx_rows, axis=0)

    b = step // (S_ // TOK_BLK)
    bmask = (
        jax.lax.broadcasted_iota(jnp.int32, (E_, B_), 1) == b
    ).astype(jnp.float32)
    chosen = jnp.where(s < 0.0, 1.0, 0.0)
    acc_ref[...] += jnp.sum(scores, axis=1, keepdims=True) * bmask
    cnt_ref[...] += jnp.sum(chosen, axis=1, keepdims=True) * bmask

    @pl.when(step == pl.num_programs(0) - 1)
    def _finalize():
        aux_ref[...] = jnp.sum(
            acc_ref[...] * cnt_ref[...], keepdims=True
        ).reshape(1, 1) * AUX_SCALE


def kernel(hidden_states, weight):
    n = B_ * S_
    x = hidden_states.reshape(n, H_)
    grid = n // TOK_BLK
    packed, aux = pl.pallas_call(
        _gate_kernel,
        grid=(grid,),
        in_specs=[
            pl.BlockSpec((TOK_BLK, H_), lambda i: (i, 0)),
            pl.BlockSpec((E_, H_), lambda i: (0, 0)),
        ],
        out_specs=[
            pl.BlockSpec((2 * K_, TOK_BLK), lambda i: (0, i)),
            pl.BlockSpec((1, 1), lambda i: (0, 0)),
        ],
        out_shape=[
            jax.ShapeDtypeStruct((2 * K_, n), jnp.float32),
            jax.ShapeDtypeStruct((1, 1), jnp.float32),
        ],
        scratch_shapes=[
            pltpu.VMEM((E_, B_), jnp.float32),
            pltpu.VMEM((E_, B_), jnp.float32),
        ],
    )(x, weight)
    pt = packed.T
    return (
        pt[:, K_:].astype(jnp.int32),
        pt[:, :K_],
        aux.reshape(()),
    )


# speedup vs baseline: 1.0616x; 1.0616x over previous
import jax
import jax.numpy as jnp
from jax.experimental import pallas as pl
from jax.experimental.pallas import tpu as pltpu

B_, S_, H_, E_, K_ = 4, 8192, 1024, 64, 8
ALPHA_ = 0.1
TOK_BLK = 4096
AUX_SCALE = E_ * ALPHA_ / (B_ * float(S_) * float(S_) * K_)


def _gate_kernel(x_ref, w_ref, idx_ref, gate_ref, aux_ref, acc_ref, cnt_ref):
    step = pl.program_id(0)

    @pl.when(step == 0)
    def _init():
        acc_ref[...] = jnp.zeros_like(acc_ref)
        cnt_ref[...] = jnp.zeros_like(cnt_ref)

    x = x_ref[...]
    w = w_ref[...]
    logits = jax.lax.dot_general(
        w, x, (((1,), (1,)), ((), ())), preferred_element_type=jnp.float32
    )
    m = jnp.max(logits, axis=0, keepdims=True)
    ex = jnp.exp(logits - m)
    z = jnp.sum(ex, axis=0, keepdims=True)
    r = 1.0 / z

    iota = jax.lax.broadcasted_iota(jnp.int32, ex.shape, 0).astype(
        jnp.float32
    )
    s = ex
    idx_rows = []
    val_rows = []
    for k in range(K_):
        if k == 0:
            vmax = jnp.full_like(z, 1.0)
        else:
            vmax = jnp.max(s, axis=0, keepdims=True)
        cand = jnp.where(s == vmax, iota, 64.0)
        imax = jnp.min(cand, axis=0, keepdims=True)
        onehot = cand == imax
        s = jnp.where(onehot, -1.0, s)
        idx_rows.append(imax)
        val_rows.append(vmax)

    vals = jnp.concatenate(val_rows, axis=0)
    denom = jnp.sum(vals, axis=0, keepdims=True) + 1e-20
    gate_ref[...] = vals / denom
    idx_ref[...] = jnp.concatenate(idx_rows, axis=0).astype(jnp.int32)

    b = step // (S_ // TOK_BLK)
    bmask = (
        jax.lax.broadcasted_iota(jnp.int32, (E_, B_), 1) == b
    ).astype(jnp.float32)
    chosen = jnp.where(s < 0.0, 1.0, 0.0)
    acc_ref[...] += jnp.sum(ex * r, axis=1, keepdims=True) * bmask
    cnt_ref[...] += jnp.sum(chosen, axis=1, keepdims=True) * bmask

    @pl.when(step == pl.num_programs(0) - 1)
    def _finalize():
        aux_ref[...] = jnp.sum(
            acc_ref[...] * cnt_ref[...], keepdims=True
        ).reshape(1, 1) * AUX_SCALE


def kernel(hidden_states, weight):
    n = B_ * S_
    x = hidden_states.reshape(n, H_)
    grid = n // TOK_BLK
    idx8, gate8, aux = pl.pallas_call(
        _gate_kernel,
        grid=(grid,),
        in_specs=[
            pl.BlockSpec((TOK_BLK, H_), lambda i: (i, 0)),
            pl.BlockSpec((E_, H_), lambda i: (0, 0)),
        ],
        out_specs=[
            pl.BlockSpec((K_, TOK_BLK), lambda i: (0, i)),
            pl.BlockSpec((K_, TOK_BLK), lambda i: (0, i)),
            pl.BlockSpec((1, 1), lambda i: (0, 0)),
        ],
        out_shape=[
            jax.ShapeDtypeStruct((K_, n), jnp.int32),
            jax.ShapeDtypeStruct((K_, n), jnp.float32),
            jax.ShapeDtypeStruct((1, 1), jnp.float32),
        ],
        scratch_shapes=[
            pltpu.VMEM((E_, B_), jnp.float32),
            pltpu.VMEM((E_, B_), jnp.float32),
        ],
    )(x, weight)
    return (idx8.T, gate8.T, aux.reshape(()))
